# baseline (speedup 1.0000x reference)
import functools

import jax
import jax.numpy as jnp
from jax import lax
from jax.experimental import pallas as pl
from jax.experimental.pallas import tpu as pltpu
from jax.experimental.pallas import tpu_sc as plsc

N_NODES = 10000
D = 128
NC = 2
NS = 16
NW = NC * NS
CH = 128
NPAD = 10240
RPT = NPAD // NS
NBUF = 2

_mesh = plsc.VectorSubcoreMesh(core_axis_name="c", subcore_axis_name="s")


def _deg_body(row_hbm, zeros1_hbm, deg_hbm, row_v, ones_v, acc, dsem):
    c = lax.axis_index("c")
    s = lax.axis_index("s")
    w = c * NS + s
    k = row_hbm.shape[1]
    rs = pl.ds(s * RPT, RPT)
    pltpu.sync_copy(zeros1_hbm.at[rs], acc.at[rs])
    for i in range(CH // 16):
        ones_v[pl.ds(i * 16, 16)] = jnp.ones((16,), jnp.float32)
    pltpu.sync_copy(row_hbm.at[w], row_v)
    plsc.subcore_barrier()

    def fire(j, carry):
        pltpu.async_copy(ones_v, acc.at[row_v.at[j]], dsem, add=True)
        return carry

    lax.fori_loop(0, k, fire, 0)

    def drain(j, carry):
        pltpu.make_async_copy(ones_v, acc.at[row_v.at[0]], dsem).wait()
        return carry

    lax.fori_loop(0, k, drain, 0)
    plsc.subcore_barrier()
    pltpu.sync_copy(acc.at[rs], deg_hbm.at[w])


def _agg_body(z_hbm, col_hbm, row_hbm, zeros2_hbm, s_hbm,
              cw0, cw1, rw0, rw1, gb0, gb1, acc,
              is0, is1, gs0, gs1, ss0, ss1):
    colw = [cw0, cw1]
    roww = [rw0, rw1]
    gbuf = [gb0, gb1]
    isem = [is0, is1]
    gsem = [gs0, gs1]
    ssem = [ss0, ss1]
    c = lax.axis_index("c")
    s = lax.axis_index("s")
    w = c * NS + s
    k = col_hbm.shape[1]
    ngroups = k // NBUF
    rs = pl.ds(s * RPT, RPT)

    @pl.when(c == 0)
    def _():
        pltpu.sync_copy(z_hbm.at[rs], acc.at[rs])

    @pl.when(c == 1)
    def _():
        pltpu.sync_copy(zeros2_hbm.at[rs], acc.at[rs])

    plsc.subcore_barrier()

    def load_idx(g, p, sem_slot):
        gs_ = pl.ds(g * NBUF, NBUF)
        pltpu.async_copy(col_hbm.at[w, gs_], colw[p], isem[sem_slot])
        pltpu.async_copy(row_hbm.at[w, gs_], roww[p], isem[sem_slot])

    def wait_idx(p, sem_slot):
        pltpu.make_async_copy(col_hbm.at[w, pl.ds(0, NBUF)], colw[p],
                              isem[sem_slot]).wait()
        pltpu.make_async_copy(row_hbm.at[w, pl.ds(0, NBUF)], roww[p],
                              isem[sem_slot]).wait()

    load_idx(0, 0, 0)
    wait_idx(0, 0)
    for b in range(NBUF):
        pltpu.async_copy(z_hbm.at[colw[0].at[b]], gbuf[b], gsem[b])
    load_idx(1, 1, 1)

    def run_group(g, p):
        pp = 1 - p
        for b in range(NBUF):
            pltpu.make_async_copy(
                z_hbm.at[colw[p].at[b]], gbuf[b], gsem[b]).wait()
            pltpu.async_copy(gbuf[b], acc.at[roww[p].at[b]], ssem[b],
                             add=True)
            pltpu.make_async_copy(
                gbuf[b], acc.at[roww[p].at[b]], ssem[b]).wait()

            if b == 0:
                @pl.when(g + 1 < ngroups)
                def _():
                    wait_idx(pp, pp)

            @pl.when(g + 1 < ngroups)
            def _():
                pltpu.async_copy(z_hbm.at[colw[pp].at[b]], gbuf[b], gsem[b])

        @pl.when(g + 2 < ngroups)
        def _():
            load_idx(g + 2, p, p)

    def pair(t, carry):
        run_group(2 * t, 0)
        run_group(2 * t + 1, 1)
        return carry

    lax.fori_loop(0, ngroups // 2, pair, 0)
    plsc.subcore_barrier()
    pltpu.sync_copy(acc.at[rs], s_hbm.at[w])


def _mm_body(x_ref, w_ref, u_ref):
    u_ref[...] = jnp.dot(x_ref[...], w_ref[...],
                         preferred_element_type=jnp.float32)


def _scale_body(u_ref, dt_ref, z_ref):
    n = u_ref.shape[0]
    d = dt_ref[0:n]
    dis = lax.rsqrt(d[:, 0:1] + d[:, 1:2] + 1.0)
    z_ref[0:n] = dis * u_ref[...]
    z_ref[n:] = jnp.zeros((NPAD - n, D), jnp.float32)


def _final_body(s_ref, dt_ref, o_ref):
    n = o_ref.shape[0]
    d = dt_ref[...]
    dis = lax.rsqrt(d[:, 0:1] + d[:, 1:2] + 1.0)
    sp = s_ref[...].reshape(NC, NPAD, D)
    o_ref[...] = dis * (sp[0, :n] + sp[1, :n])


def kernel(x, edge_index, weight):
    n, d_in = x.shape
    e = edge_index.shape[1]
    k = -(-e // (NW * CH))
    k = -(-k // (2 * NBUF)) * (2 * NBUF)
    ep = NW * k * CH

    row = edge_index[0].astype(jnp.int32)
    col = edge_index[1].astype(jnp.int32)
    pad = (n + (jnp.arange(ep - e, dtype=jnp.int32) % (NPAD - n)))
    row3d = jnp.concatenate([row, pad]).reshape(NW, k, CH)
    col3d = jnp.concatenate([col, pad]).reshape(NW, k, CH)

    zeros1 = jnp.zeros((NPAD,), jnp.float32)
    zeros2 = jnp.zeros((NPAD, D), jnp.float32)

    u = pl.pallas_call(
        _mm_body,
        out_shape=jax.ShapeDtypeStruct((n, D), jnp.float32),
    )(x, weight)

    deg_kernel = functools.partial(
        pl.kernel, mesh=_mesh,
        out_type=jax.ShapeDtypeStruct((NW, RPT), jnp.float32),
        scratch_types=[
            pltpu.VMEM((k, CH), jnp.int32),
            pltpu.VMEM((CH,), jnp.float32),
            pltpu.VMEM_SHARED((NPAD,), jnp.float32),
            pltpu.SemaphoreType.DMA,
        ])(_deg_body)
    deg_parts = deg_kernel(row3d, zeros1)
    deg_t = deg_parts.reshape(NC, NPAD).T

    z_pad = pl.pallas_call(
        _scale_body,
        out_shape=jax.ShapeDtypeStruct((NPAD, D), jnp.float32),
    )(u, deg_t)

    agg_kernel = functools.partial(
        pl.kernel, mesh=_mesh,
        out_type=jax.ShapeDtypeStruct((NW, RPT, D), jnp.float32),
        scratch_types=(
            [pltpu.VMEM((NBUF, CH), jnp.int32)] * 4
            + [pltpu.VMEM((CH, D), jnp.float32)] * NBUF
            + [pltpu.VMEM_SHARED((NPAD, D), jnp.float32)]
            + [pltpu.SemaphoreType.DMA] * 6
        ))(_agg_body)
    s_parts = agg_kernel(z_pad, col3d, row3d, zeros2)

    out = pl.pallas_call(
        _final_body,
        out_shape=jax.ShapeDtypeStruct((n, D), jnp.float32),
    )(s_parts, deg_t[:n])
    return out


# speedup vs baseline: 1.0051x; 1.0051x over previous
import functools

import jax
import jax.numpy as jnp
from jax import lax
from jax.experimental import pallas as pl
from jax.experimental.pallas import tpu as pltpu
from jax.experimental.pallas import tpu_sc as plsc

N_NODES = 10000
D = 128
NC = 2
NS = 16
NW = NC * NS
CH = 128
NPAD = 10240
RPT = NPAD // NS
NBUF = 2

_mesh = plsc.VectorSubcoreMesh(core_axis_name="c", subcore_axis_name="s")


def _deg_body(row_hbm, zeros1_hbm, deg_hbm, row_v, ones_v, acc, dsem):
    c = lax.axis_index("c")
    s = lax.axis_index("s")
    w = c * NS + s
    k = row_hbm.shape[1]
    rs = pl.ds(s * RPT, RPT)
    pltpu.sync_copy(zeros1_hbm.at[rs], acc.at[rs])
    for i in range(CH // 16):
        ones_v[pl.ds(i * 16, 16)] = jnp.ones((16,), jnp.float32)
    pltpu.sync_copy(row_hbm.at[w], row_v)
    plsc.subcore_barrier()

    def fire(j, carry):
        pltpu.async_copy(ones_v, acc.at[row_v.at[j]], dsem, add=True)
        return carry

    lax.fori_loop(0, k, fire, 0)

    def drain(j, carry):
        pltpu.make_async_copy(ones_v, acc.at[row_v.at[0]], dsem).wait()
        return carry

    lax.fori_loop(0, k, drain, 0)
    plsc.subcore_barrier()
    pltpu.sync_copy(acc.at[rs], deg_hbm.at[w])


def _agg_body(z_hbm, col_hbm, row_hbm, zeros2_hbm, s_hbm,
              cw0, cw1, rw0, rw1, gb0, gb1, acc,
              is0, is1, gs0, gs1, ss0, ss1):
    colw = [cw0, cw1]
    roww = [rw0, rw1]
    gbuf = [gb0, gb1]
    isem = [is0, is1]
    gsem = [gs0, gs1]
    ssem = [ss0, ss1]
    c = lax.axis_index("c")
    s = lax.axis_index("s")
    w = c * NS + s
    k = col_hbm.shape[1]
    ngroups = k // NBUF
    rs = pl.ds(s * RPT, RPT)

    @pl.when(c == 0)
    def _():
        pltpu.sync_copy(z_hbm.at[rs], acc.at[rs])

    @pl.when(c == 1)
    def _():
        pltpu.sync_copy(zeros2_hbm.at[rs], acc.at[rs])

    plsc.subcore_barrier()

    def load_idx(g, p, sem_slot):
        gs_ = pl.ds(g * NBUF, NBUF)
        pltpu.async_copy(col_hbm.at[w, gs_], colw[p], isem[sem_slot])
        pltpu.async_copy(row_hbm.at[w, gs_], roww[p], isem[sem_slot])

    def wait_idx(p, sem_slot):
        pltpu.make_async_copy(col_hbm.at[w, pl.ds(0, NBUF)], colw[p],
                              isem[sem_slot]).wait()
        pltpu.make_async_copy(row_hbm.at[w, pl.ds(0, NBUF)], roww[p],
                              isem[sem_slot]).wait()

    load_idx(0, 0, 0)
    wait_idx(0, 0)
    for b in range(NBUF):
        pltpu.async_copy(z_hbm.at[colw[0].at[b]], gbuf[b], gsem[b])
    load_idx(1, 1, 1)

    def run_group(g, p):
        pp = 1 - p
        for b in range(NBUF):
            pltpu.make_async_copy(
                z_hbm.at[colw[p].at[b]], gbuf[b], gsem[b]).wait()
            pltpu.async_copy(gbuf[b], acc.at[roww[p].at[b]], ssem[b],
                             add=True)
            pltpu.make_async_copy(
                gbuf[b], acc.at[roww[p].at[b]], ssem[b]).wait()

            if b == 0:
                @pl.when(g + 1 < ngroups)
                def _():
                    wait_idx(pp, pp)

            @pl.when(g + 1 < ngroups)
            def _():
                pltpu.async_copy(z_hbm.at[colw[pp].at[b]], gbuf[b], gsem[b])

        @pl.when(g + 2 < ngroups)
        def _():
            load_idx(g + 2, p, p)

    def pair(t, carry):
        run_group(2 * t, 0)
        run_group(2 * t + 1, 1)
        return carry

    lax.fori_loop(0, ngroups // 2, pair, 0)
    plsc.subcore_barrier()
    pltpu.sync_copy(acc.at[rs], s_hbm.at[w])


def _proj_body(x_ref, w_ref, dt_ref, z_ref):
    n = x_ref.shape[0]
    d = dt_ref[0:n]
    dis = lax.rsqrt(d[:, 0:1] + d[:, 1:2] + 1.0)
    z_ref[0:n] = dis * jnp.dot(x_ref[...], w_ref[...],
                               preferred_element_type=jnp.float32)
    z_ref[n:] = jnp.zeros((NPAD - n, D), jnp.float32)


def _final_body(s_ref, dt_ref, o_ref):
    n = o_ref.shape[0]
    d = dt_ref[...]
    dis = lax.rsqrt(d[:, 0:1] + d[:, 1:2] + 1.0)
    sp = s_ref[...].reshape(NC, NPAD, D)
    o_ref[...] = dis * (sp[0, :n] + sp[1, :n])


def kernel(x, edge_index, weight):
    n, d_in = x.shape
    e = edge_index.shape[1]
    k = -(-e // (NW * CH))
    k = -(-k // (2 * NBUF)) * (2 * NBUF)
    ep = NW * k * CH

    row = edge_index[0].astype(jnp.int32)
    col = edge_index[1].astype(jnp.int32)
    pad = (n + (jnp.arange(ep - e, dtype=jnp.int32) % (NPAD - n)))
    row3d = jnp.concatenate([row, pad]).reshape(NW, k, CH)
    col3d = jnp.concatenate([col, pad]).reshape(NW, k, CH)

    zeros1 = jnp.zeros((NPAD,), jnp.float32)
    zeros2 = jnp.zeros((NPAD, D), jnp.float32)

    deg_kernel = functools.partial(
        pl.kernel, mesh=_mesh,
        out_type=jax.ShapeDtypeStruct((NW, RPT), jnp.float32),
        scratch_types=[
            pltpu.VMEM((k, CH), jnp.int32),
            pltpu.VMEM((CH,), jnp.float32),
            pltpu.VMEM_SHARED((NPAD,), jnp.float32),
            pltpu.SemaphoreType.DMA,
        ])(_deg_body)
    deg_parts = deg_kernel(row3d, zeros1)
    deg_t = deg_parts.reshape(NC, NPAD).T

    z_pad = pl.pallas_call(
        _proj_body,
        out_shape=jax.ShapeDtypeStruct((NPAD, D), jnp.float32),
    )(x, weight, deg_t)

    agg_kernel = functools.partial(
        pl.kernel, mesh=_mesh,
        out_type=jax.ShapeDtypeStruct((NW, RPT, D), jnp.float32),
        scratch_types=(
            [pltpu.VMEM((NBUF, CH), jnp.int32)] * 4
            + [pltpu.VMEM((CH, D), jnp.float32)] * NBUF
            + [pltpu.VMEM_SHARED((NPAD, D), jnp.float32)]
            + [pltpu.SemaphoreType.DMA] * 6
        ))(_agg_body)
    s_parts = agg_kernel(z_pad, col3d, row3d, zeros2)

    out = pl.pallas_call(
        _final_body,
        out_shape=jax.ShapeDtypeStruct((n, D), jnp.float32),
    )(s_parts, deg_t[:n])
    return out


# speedup vs baseline: 1.0221x; 1.0168x over previous
import functools

import jax
import jax.numpy as jnp
from jax import lax
from jax.experimental import pallas as pl
from jax.experimental.pallas import tpu as pltpu
from jax.experimental.pallas import tpu_sc as plsc

N_NODES = 10000
D = 128
NC = 2
NS = 16
NW = NC * NS
CH = 128
NPAD = 10240
RPT = NPAD // NS
NBUF = 2

_mesh = plsc.VectorSubcoreMesh(core_axis_name="c", subcore_axis_name="s")


def _deg_body(row_hbm, zeros1_hbm, deg_hbm, row_v, ones_v, acc, dsem):
    c = lax.axis_index("c")
    s = lax.axis_index("s")
    w = c * NS + s
    k = row_hbm.shape[1]
    rs = pl.ds(s * RPT, RPT)
    pltpu.sync_copy(zeros1_hbm.at[rs], acc.at[rs])
    for i in range(CH // 16):
        ones_v[pl.ds(i * 16, 16)] = jnp.ones((16,), jnp.float32)
    pltpu.sync_copy(row_hbm.at[w], row_v)
    plsc.subcore_barrier()

    def fire(j, carry):
        pltpu.async_copy(ones_v, acc.at[row_v.at[j]], dsem, add=True)
        return carry

    lax.fori_loop(0, k, fire, 0)

    def drain(j, carry):
        pltpu.make_async_copy(ones_v, acc.at[row_v.at[0]], dsem).wait()
        return carry

    lax.fori_loop(0, k, drain, 0)
    plsc.subcore_barrier()
    pltpu.sync_copy(acc.at[rs], deg_hbm.at[w])


def _agg_body(z_hbm, col_hbm, row_hbm, zeros2_hbm, s_hbm,
              cw0, cw1, rw0, rw1, gb0, gb1, acc,
              is0, is1, gs0, gs1, ss0, ss1, nsem):
    colw = [cw0, cw1]
    roww = [rw0, rw1]
    gbuf = [gb0, gb1]
    isem = [is0, is1]
    gsem = [gs0, gs1]
    ssem = [ss0, ss1]
    c = lax.axis_index("c")
    s = lax.axis_index("s")
    w = c * NS + s
    k = col_hbm.shape[1]
    ngroups = k // NBUF
    rs = pl.ds(s * RPT, RPT)

    @pl.when(c == 0)
    def _():
        pltpu.async_copy(z_hbm.at[rs], acc.at[rs], nsem)

    @pl.when(c == 1)
    def _():
        pltpu.async_copy(zeros2_hbm.at[rs], acc.at[rs], nsem)

    def load_idx(g, p, sem_slot):
        gs_ = pl.ds(g * NBUF, NBUF)
        pltpu.async_copy(col_hbm.at[w, gs_], colw[p], isem[sem_slot])
        pltpu.async_copy(row_hbm.at[w, gs_], roww[p], isem[sem_slot])

    def wait_idx(p, sem_slot):
        pltpu.make_async_copy(col_hbm.at[w, pl.ds(0, NBUF)], colw[p],
                              isem[sem_slot]).wait()
        pltpu.make_async_copy(row_hbm.at[w, pl.ds(0, NBUF)], roww[p],
                              isem[sem_slot]).wait()

    load_idx(0, 0, 0)
    wait_idx(0, 0)
    for b in range(NBUF):
        pltpu.async_copy(z_hbm.at[colw[0].at[b]], gbuf[b], gsem[b])
    load_idx(1, 1, 1)
    pltpu.make_async_copy(z_hbm.at[rs], acc.at[rs], nsem).wait()
    plsc.subcore_barrier()

    def run_group(g, p):
        pp = 1 - p
        for b in range(NBUF):
            pltpu.make_async_copy(
                z_hbm.at[colw[p].at[b]], gbuf[b], gsem[b]).wait()
            pltpu.async_copy(gbuf[b], acc.at[roww[p].at[b]], ssem[b],
                             add=True)
            pltpu.make_async_copy(
                gbuf[b], acc.at[roww[p].at[b]], ssem[b]).wait()

            if b == 0:
                @pl.when(g + 1 < ngroups)
                def _():
                    wait_idx(pp, pp)

            @pl.when(g + 1 < ngroups)
            def _():
                pltpu.async_copy(z_hbm.at[colw[pp].at[b]], gbuf[b], gsem[b])

        @pl.when(g + 2 < ngroups)
        def _():
            load_idx(g + 2, p, p)

    def pair(t, carry):
        run_group(2 * t, 0)
        run_group(2 * t + 1, 1)
        return carry

    lax.fori_loop(0, ngroups // 2, pair, 0)
    plsc.subcore_barrier()
    pltpu.sync_copy(acc.at[rs], s_hbm.at[w])


def _proj_body(x_ref, w_ref, dt_ref, z_ref):
    n = x_ref.shape[0]
    d = dt_ref[0:n]
    dis = lax.rsqrt(d[:, 0:1] + d[:, 1:2] + 1.0)
    z_ref[0:n] = dis * jnp.dot(x_ref[...], w_ref[...],
                               preferred_element_type=jnp.float32)
    z_ref[n:] = jnp.zeros((NPAD - n, D), jnp.float32)


def _final_body(s_ref, dt_ref, o_ref):
    n = o_ref.shape[0]
    d = dt_ref[...]
    dis = lax.rsqrt(d[:, 0:1] + d[:, 1:2] + 1.0)
    sp = s_ref[...].reshape(NC, NPAD, D)
    o_ref[...] = dis * (sp[0, :n] + sp[1, :n])


def kernel(x, edge_index, weight):
    n, d_in = x.shape
    e = edge_index.shape[1]
    k = -(-e // (NW * CH))
    k = -(-k // (2 * NBUF)) * (2 * NBUF)
    ep = NW * k * CH

    row = edge_index[0].astype(jnp.int32)
    col = edge_index[1].astype(jnp.int32)
    pad = (n + (jnp.arange(ep - e, dtype=jnp.int32) % (NPAD - n)))
    row3d = jnp.concatenate([row, pad]).reshape(NW, k, CH)
    col3d = jnp.concatenate([col, pad]).reshape(NW, k, CH)

    zeros1 = jnp.zeros((NPAD,), jnp.float32)
    zeros2 = jnp.zeros((NPAD, D), jnp.float32)

    deg_kernel = functools.partial(
        pl.kernel, mesh=_mesh,
        out_type=jax.ShapeDtypeStruct((NW, RPT), jnp.float32),
        scratch_types=[
            pltpu.VMEM((k, CH), jnp.int32),
            pltpu.VMEM((CH,), jnp.float32),
            pltpu.VMEM_SHARED((NPAD,), jnp.float32),
            pltpu.SemaphoreType.DMA,
        ])(_deg_body)
    deg_parts = deg_kernel(row3d, zeros1)
    deg_t = deg_parts.reshape(NC, NPAD).T

    z_pad = pl.pallas_call(
        _proj_body,
        out_shape=jax.ShapeDtypeStruct((NPAD, D), jnp.float32),
    )(x, weight, deg_t)

    agg_kernel = functools.partial(
        pl.kernel, mesh=_mesh,
        out_type=jax.ShapeDtypeStruct((NW, RPT, D), jnp.float32),
        scratch_types=(
            [pltpu.VMEM((NBUF, CH), jnp.int32)] * 4
            + [pltpu.VMEM((CH, D), jnp.float32)] * NBUF
            + [pltpu.VMEM_SHARED((NPAD, D), jnp.float32)]
            + [pltpu.SemaphoreType.DMA] * 7
        ))(_agg_body)
    s_parts = agg_kernel(z_pad, col3d, row3d, zeros2)

    out = pl.pallas_call(
        _final_body,
        out_shape=jax.ShapeDtypeStruct((n, D), jnp.float32),
    )(s_parts, deg_t[:n])
    return out
